# baseline (speedup 1.0000x reference)
import functools

import jax
import jax.numpy as jnp
from jax import lax
from jax.experimental import pallas as pl
from jax.experimental.pallas import tpu as pltpu
from jax.experimental.pallas import tpu_sc as plsc

_LN_EPS = 1e-5
_BN_EPS = 1e-5
_CH = 125
_NPAD = 10240


def _sc_agg_body(h2, srcg, dstl, out, src_v, dst_v, rows_v, acc, sem,
                 *, n, rpt, tpb):
    c = lax.axis_index("c")
    s = lax.axis_index("s")
    pltpu.sync_copy(h2.at[pl.ds(c * n + s * tpb, tpb)],
                    acc.at[pl.ds(s * tpb, tpb)])
    base = s * rpt
    pltpu.sync_copy(srcg.at[c, pl.ds(base, rpt)], src_v)
    pltpu.sync_copy(dstl.at[pl.ds(base, rpt)], dst_v)
    plsc.subcore_barrier()

    def body(j, carry):
        pltpu.async_copy(h2.at[src_v.at[j]], rows_v, sem).wait()
        pltpu.sync_copy(rows_v, acc.at[dst_v.at[j]], add=True)
        return carry

    lax.fori_loop(0, rpt, body, 0)
    plsc.subcore_barrier()
    pltpu.sync_copy(acc.at[pl.ds(s * tpb, tpb)],
                    out.at[pl.ds(c * n + s * tpb, tpb)])


def _sc_agg(h2f, srcg, dstl, *, n):
    rows, ch = srcg.shape[1], srcg.shape[2]
    rpt = rows // 16
    tpb = n // 16
    f = pl.kernel(
        functools.partial(_sc_agg_body, n=n, rpt=rpt, tpb=tpb),
        out_type=jax.ShapeDtypeStruct((2 * n, 128), jnp.float32),
        mesh=plsc.VectorSubcoreMesh(core_axis_name="c", subcore_axis_name="s"),
        scratch_types=[
            pltpu.VMEM((rpt, ch), jnp.int32),
            pltpu.VMEM((rpt, ch), jnp.int32),
            pltpu.VMEM((ch, 128), jnp.float32),
            pltpu.VMEM_SHARED((n, 128), jnp.float32),
            pltpu.SemaphoreType.DMA,
        ],
    )
    return f(h2f, srcg, dstl)


def _sc_policy_body(a_hbm, b_hbm, srcp, dstp, w_hbm, out, src_v, dst_v,
                    abuf0, bbuf0, abuf1, bbuf1, out_v, w_v,
                    sa0, sb0, sa1, sb1, *, chunks, ch):
    c = lax.axis_index("c")
    s = lax.axis_index("s")
    wid = c * 16 + s
    base = wid * chunks
    pltpu.sync_copy(srcp.at[pl.ds(base, chunks)], src_v)
    pltpu.sync_copy(dstp.at[pl.ds(base, chunks)], dst_v)
    pltpu.sync_copy(w_hbm, w_v)

    def compute(ab, bb):
        groups = ch // 16
        ridx = [lax.broadcasted_iota(jnp.int32, (16,), 0) + g * 16
                for g in range(groups)]
        zero = jnp.zeros((16,), jnp.float32)

        def dbody(t, accs):
            w16 = w_v[pl.ds(t * 16, 16)]
            accs = list(accs)
            for i in range(16):
                dd = t * 16 + i
                cidx = jnp.full((16,), dd, jnp.int32)
                for g in range(groups):
                    pe = jnp.maximum(
                        plsc.load_gather(ab, [ridx[g], cidx])
                        + plsc.load_gather(bb, [ridx[g], cidx]), 0.0)
                    u = plsc.bitcast(pe, jnp.uint32)
                    u = (u + 0x7FFF + ((u >> 16) & 1)) & jnp.uint32(0xFFFF0000)
                    accs[g] = accs[g] + plsc.bitcast(u, jnp.float32) * w16[i]
            return tuple(accs)

        accs = lax.fori_loop(0, 16, dbody, tuple(zero for _ in range(groups)))
        for g in range(groups):
            out_v[pl.ds(g * 16, 16)] = accs[g]

    pltpu.async_copy(a_hbm.at[src_v.at[0]], abuf0, sa0)
    pltpu.async_copy(b_hbm.at[dst_v.at[0]], bbuf0, sb0)

    def body(jj, carry):
        j = 2 * jj
        c1 = pltpu.async_copy(a_hbm.at[src_v.at[j + 1]], abuf1, sa1)
        c2 = pltpu.async_copy(b_hbm.at[dst_v.at[j + 1]], bbuf1, sb1)
        pltpu.make_async_copy(a_hbm.at[src_v.at[j]], abuf0, sa0).wait()
        pltpu.make_async_copy(b_hbm.at[dst_v.at[j]], bbuf0, sb0).wait()
        compute(abuf0, bbuf0)
        pltpu.sync_copy(out_v, out.at[base + j])

        @pl.when(jj + 1 < chunks // 2)
        def _():
            pltpu.async_copy(a_hbm.at[src_v.at[j + 2]], abuf0, sa0)
            pltpu.async_copy(b_hbm.at[dst_v.at[j + 2]], bbuf0, sb0)

        c1.wait()
        c2.wait()
        compute(abuf1, bbuf1)
        pltpu.sync_copy(out_v, out.at[base + j + 1])
        return carry

    lax.fori_loop(0, chunks // 2, body, 0)


def _sc_policy(a_nodes, b_nodes, srcp, dstp, w2):
    rows, ch = srcp.shape
    chunks = rows // 32
    f = pl.kernel(
        functools.partial(_sc_policy_body, chunks=chunks, ch=ch),
        out_type=jax.ShapeDtypeStruct((rows, ch), jnp.float32),
        mesh=plsc.VectorSubcoreMesh(core_axis_name="c", subcore_axis_name="s"),
        compiler_params=pltpu.CompilerParams(needs_layout_passes=False),
        scratch_types=[
            pltpu.VMEM((chunks, ch), jnp.int32),
            pltpu.VMEM((chunks, ch), jnp.int32),
            pltpu.VMEM((ch, 256), jnp.float32),
            pltpu.VMEM((ch, 256), jnp.float32),
            pltpu.VMEM((ch, 256), jnp.float32),
            pltpu.VMEM((ch, 256), jnp.float32),
            pltpu.VMEM((ch,), jnp.float32),
            pltpu.VMEM((256,), jnp.float32),
            pltpu.SemaphoreType.DMA,
            pltpu.SemaphoreType.DMA,
            pltpu.SemaphoreType.DMA,
            pltpu.SemaphoreType.DMA,
        ],
    )
    return f(a_nodes, b_nodes, srcp, dstp, w2)


def _gin_dense_body(s3, w, b, g, beta, out3, *, relu):
    s = jnp.concatenate([s3[0], s3[1]], axis=-1)
    h = lax.dot_general(s, w[...], (((1,), (1,)), ((), ())),
                        preferred_element_type=jnp.float32) + b[...]
    mu = jnp.mean(h, axis=-1, keepdims=True)
    var = jnp.mean((h - mu) ** 2, axis=-1, keepdims=True)
    h = (h - mu) * lax.rsqrt(var + _LN_EPS) * g[...] + beta[...]
    if relu:
        h = jnp.maximum(h, 0.0)
    out3[0] = h[:, :128]
    out3[1] = h[:, 128:]


def _gin_dense(s3, w, b, g, beta, *, nreal, relu, bn=1000):
    n = s3.shape[1]
    d = 2 * s3.shape[2]
    nb = nreal // bn
    return pl.pallas_call(
        functools.partial(_gin_dense_body, relu=relu),
        grid=(nb,),
        in_specs=[
            pl.BlockSpec((2, bn, d // 2), lambda i: (0, i, 0)),
            pl.BlockSpec((d, d), lambda i: (0, 0)),
            pl.BlockSpec((1, d), lambda i: (0, 0)),
            pl.BlockSpec((1, d), lambda i: (0, 0)),
            pl.BlockSpec((1, d), lambda i: (0, 0)),
        ],
        out_specs=pl.BlockSpec((2, bn, d // 2), lambda i: (0, i, 0)),
        out_shape=jax.ShapeDtypeStruct((2, n, d // 2), jnp.float32),
        compiler_params=pltpu.CompilerParams(
            dimension_semantics=("arbitrary",)),
    )(s3, w, b.reshape(1, d), g.reshape(1, d), beta.reshape(1, d))


def _finale_body(s3, xres3, w, b, g, beta, p1a, p1b, p1bias, batch_r,
                 a_out, b_out, pool_out, cnt_out, *, nseg):
    i = pl.program_id(0)
    s = jnp.concatenate([s3[0], s3[1]], axis=-1)
    res = jnp.concatenate([xres3[0], xres3[1]], axis=-1)
    h = lax.dot_general(s, w[...], (((1,), (1,)), ((), ())),
                        preferred_element_type=jnp.float32) + b[...]
    mu = jnp.mean(h, axis=-1, keepdims=True)
    var = jnp.mean((h - mu) ** 2, axis=-1, keepdims=True)
    h = (h - mu) * lax.rsqrt(var + _LN_EPS) * g[...] + beta[...]
    h = jnp.maximum(h + res, 0.0)
    a_out[...] = lax.dot_general(
        h, p1a[...], (((1,), (1,)), ((), ())),
        preferred_element_type=jnp.float32) + p1bias[...]
    b_out[...] = lax.dot_general(
        h, p1b[...], (((1,), (1,)), ((), ())),
        preferred_element_type=jnp.float32)
    bid = batch_r[0, 0, :]
    oh = (bid[:, None] == lax.broadcasted_iota(jnp.int32, (bid.shape[0], nseg), 1)
          ).astype(jnp.float32)
    poolc = lax.dot_general(oh, h, (((0,), (0,)), ((), ())),
                            preferred_element_type=jnp.float32,
                            precision=lax.Precision.HIGHEST)
    cntc = jnp.broadcast_to(jnp.sum(oh, axis=0)[:, None], poolc.shape)

    @pl.when(i == 0)
    def _init():
        pool_out[...] = jnp.zeros_like(pool_out)
        cnt_out[...] = jnp.zeros_like(cnt_out)

    pool_out[...] += poolc
    cnt_out[...] += cntc


def _finale(s3, xres3, w, b, g, beta, p1a, p1b, p1bias, batch,
            *, nseg, nreal, bn=1000):
    d = 2 * s3.shape[2]
    nb = nreal // bn
    batch_r = batch.reshape(nb, 1, bn)
    return pl.pallas_call(
        functools.partial(_finale_body, nseg=nseg),
        grid=(nb,),
        in_specs=[
            pl.BlockSpec((2, bn, d // 2), lambda i: (0, i, 0)),
            pl.BlockSpec((2, bn, d // 2), lambda i: (0, i, 0)),
            pl.BlockSpec((d, d), lambda i: (0, 0)),
            pl.BlockSpec((1, d), lambda i: (0, 0)),
            pl.BlockSpec((1, d), lambda i: (0, 0)),
            pl.BlockSpec((1, d), lambda i: (0, 0)),
            pl.BlockSpec((d, d), lambda i: (0, 0)),
            pl.BlockSpec((d, d), lambda i: (0, 0)),
            pl.BlockSpec((1, d), lambda i: (0, 0)),
            pl.BlockSpec((1, 1, bn), lambda i: (i, 0, 0)),
        ],
        out_specs=[
            pl.BlockSpec((bn, d), lambda i: (i, 0)),
            pl.BlockSpec((bn, d), lambda i: (i, 0)),
            pl.BlockSpec((nseg, d), lambda i: (0, 0)),
            pl.BlockSpec((nseg, d), lambda i: (0, 0)),
        ],
        out_shape=[
            jax.ShapeDtypeStruct((nreal, d), jnp.float32),
            jax.ShapeDtypeStruct((nreal, d), jnp.float32),
            jax.ShapeDtypeStruct((nseg, d), jnp.float32),
            jax.ShapeDtypeStruct((nseg, d), jnp.float32),
        ],
        compiler_params=pltpu.CompilerParams(
            dimension_semantics=("arbitrary",)),
    )(s3, xres3, w, b.reshape(1, d), g.reshape(1, d), beta.reshape(1, d),
      p1a, p1b, p1bias.reshape(1, d), batch_r)


def _value_body(pool, cnt, w1, b1, g1, be1, w2, b2, g2, be2, vw, vb, out):
    ge = pool[...] / jnp.maximum(cnt[...], 1.0)

    def fc_bn_relu(v, w, b, g, be):
        v = lax.dot_general(v, w[...], (((1,), (1,)), ((), ())),
                            preferred_element_type=jnp.float32) + b[...]
        mu = jnp.mean(v, axis=0, keepdims=True)
        var = jnp.mean((v - mu) ** 2, axis=0, keepdims=True)
        v = (v - mu) * lax.rsqrt(var + _BN_EPS) * g[...] + be[...]
        return jnp.maximum(v, 0.0)

    v = fc_bn_relu(ge, w1, b1, g1, be1)
    v = fc_bn_relu(v, w2, b2, g2, be2)
    vbf = v.astype(jnp.bfloat16).astype(jnp.float32)
    wbf = vw[...].astype(jnp.bfloat16).astype(jnp.float32)
    val = jnp.tanh(jnp.sum(vbf * wbf, axis=-1, keepdims=True) + vb[0, 0])
    out[...] = jnp.broadcast_to(val, out.shape)


def _value_head(pool, cnt, w1, b1, g1, be1, w2, b2, g2, be2, vw, vb):
    nseg, d = pool.shape
    full = lambda s: pl.BlockSpec(s, lambda: tuple(0 for _ in s))
    out = pl.pallas_call(
        _value_body,
        in_specs=[
            full((nseg, d)), full((nseg, d)),
            full((d, d)), full((1, d)), full((1, d)), full((1, d)),
            full((d, d)), full((1, d)), full((1, d)), full((1, d)),
            full((1, d)), full((1, 1)),
        ],
        out_specs=full((nseg, 128)),
        out_shape=jax.ShapeDtypeStruct((nseg, 128), jnp.float32),
    )(pool, cnt, w1, b1.reshape(1, d), g1.reshape(1, d), be1.reshape(1, d),
      w2, b2.reshape(1, d), g2.reshape(1, d), be2.reshape(1, d),
      vw, vb.reshape(1, 1))
    return out[:, :1]


def _to_split(h):
    n, d = h.shape
    return h.reshape(n, 2, d // 2).transpose(1, 0, 2)


def kernel(x, gin1_W, gin1_b, ln1_g, ln1_b, gin2_W, gin2_b, ln2_g, ln2_b,
           gin3_W, gin3_b, ln3_g, ln3_b, fc1_W, fc1_b, bn1_g, bn1_b,
           fc2_W, fc2_b, bn2_g, bn2_b, pe1_W, pe1_b, pe2_W, pe2_b,
           vh_W, vh_b, edge_index, batch):
    n, d = x.shape
    e = edge_index.shape[1]
    nseg = 64
    src = edge_index[0]
    dst = edge_index[1]

    rows = e // _CH
    srcg = jnp.stack([src, src + _NPAD]).reshape(2, rows, _CH)
    dstl = dst.reshape(rows, _CH)

    def agg(h3):
        s2f = _sc_agg(h3.reshape(2 * _NPAD, d // 2), srcg, dstl, n=_NPAD)
        return s2f.reshape(2, _NPAD, d // 2)

    x3 = jnp.zeros((2, _NPAD, d // 2), jnp.float32).at[:, :n].set(_to_split(x))
    h3 = _gin_dense(agg(x3), gin1_W, gin1_b, ln1_g, ln1_b, nreal=n, relu=True)
    h3 = _gin_dense(agg(h3), gin2_W, gin2_b, ln2_g, ln2_b, nreal=n, relu=True)
    p1a = pe1_W[:, :d]
    p1b = pe1_W[:, d:]
    a_nodes, b_nodes, pool, cnt = _finale(
        agg(h3), x3, gin3_W, gin3_b, ln3_g, ln3_b,
        p1a, p1b, pe1_b, batch, nseg=nseg, nreal=n)

    value = _value_head(pool, cnt, fc1_W, fc1_b, bn1_g, bn1_b,
                        fc2_W, fc2_b, bn2_g, bn2_b, vh_W, vh_b)

    ch2 = 64
    epad = 32 * 80 * ch2
    srcp = jnp.zeros((epad,), jnp.int32).at[:e].set(src).reshape(epad // ch2, ch2)
    dstp = jnp.zeros((epad,), jnp.int32).at[:e].set(dst).reshape(epad // ch2, ch2)
    logits_pad = _sc_policy(a_nodes, b_nodes, srcp, dstp, pe2_W[0])
    policy_logits = logits_pad.reshape(-1)[:e] + pe2_b[0]
    return (policy_logits, value)


# speedup vs baseline: 2.6087x; 2.6087x over previous
import functools

import jax
import jax.numpy as jnp
from jax import lax
from jax.experimental import pallas as pl
from jax.experimental.pallas import tpu as pltpu
from jax.experimental.pallas import tpu_sc as plsc

_LN_EPS = 1e-5
_BN_EPS = 1e-5
_CH = 125
_NPAD = 10240


def _sc_agg_body(h2, srcg, dstl, out, src_v, dst_v, rows_v, acc, sem,
                 *, n, rpt, tpb):
    c = lax.axis_index("c")
    s = lax.axis_index("s")
    pltpu.sync_copy(h2.at[pl.ds(c * n + s * tpb, tpb)],
                    acc.at[pl.ds(s * tpb, tpb)])
    base = s * rpt
    pltpu.sync_copy(srcg.at[c, pl.ds(base, rpt)], src_v)
    pltpu.sync_copy(dstl.at[pl.ds(base, rpt)], dst_v)
    plsc.subcore_barrier()

    def body(j, carry):
        pltpu.async_copy(h2.at[src_v.at[j]], rows_v, sem).wait()
        pltpu.sync_copy(rows_v, acc.at[dst_v.at[j]], add=True)
        return carry

    lax.fori_loop(0, rpt, body, 0)
    plsc.subcore_barrier()
    pltpu.sync_copy(acc.at[pl.ds(s * tpb, tpb)],
                    out.at[pl.ds(c * n + s * tpb, tpb)])


def _sc_agg(h2f, srcg, dstl, *, n):
    rows, ch = srcg.shape[1], srcg.shape[2]
    rpt = rows // 16
    tpb = n // 16
    f = pl.kernel(
        functools.partial(_sc_agg_body, n=n, rpt=rpt, tpb=tpb),
        out_type=jax.ShapeDtypeStruct((2 * n, 128), jnp.float32),
        mesh=plsc.VectorSubcoreMesh(core_axis_name="c", subcore_axis_name="s"),
        scratch_types=[
            pltpu.VMEM((rpt, ch), jnp.int32),
            pltpu.VMEM((rpt, ch), jnp.int32),
            pltpu.VMEM((ch, 128), jnp.float32),
            pltpu.VMEM_SHARED((n, 128), jnp.float32),
            pltpu.SemaphoreType.DMA,
        ],
    )
    return f(h2f, srcg, dstl)


def _sc_policy_body(a_hbm, b_hbm, srcp, dstp, w_hbm, out, src_v, dst_v,
                    abuf0, bbuf0, abuf1, bbuf1, out_v, w_v,
                    sa0, sb0, sa1, sb1, *, chunks, ch):
    c = lax.axis_index("c")
    s = lax.axis_index("s")
    wid = c * 16 + s
    base = wid * chunks
    pltpu.sync_copy(srcp.at[pl.ds(base, chunks)], src_v)
    pltpu.sync_copy(dstp.at[pl.ds(base, chunks)], dst_v)
    pltpu.sync_copy(w_hbm, w_v)

    wv = [w_v[pl.ds(k * 16, 16)] for k in range(16)]
    lane = lax.broadcasted_iota(jnp.int32, (16,), 0)
    zero = jnp.zeros((16,), jnp.float32)

    def compute(ab, bb):
        groups = ch // 16

        def edge_dot(e):
            acc0 = zero
            acc1 = zero
            for k in range(0, 16, 2):
                pe0 = jnp.maximum(ab[e, pl.ds(k * 16, 16)]
                                  + bb[e, pl.ds(k * 16, 16)], 0.0)
                pe1 = jnp.maximum(ab[e, pl.ds(k * 16 + 16, 16)]
                                  + bb[e, pl.ds(k * 16 + 16, 16)], 0.0)
                u0 = plsc.bitcast(pe0, jnp.uint32)
                u0 = (u0 + 0x7FFF + ((u0 >> 16) & 1)) & jnp.uint32(0xFFFF0000)
                u1 = plsc.bitcast(pe1, jnp.uint32)
                u1 = (u1 + 0x7FFF + ((u1 >> 16) & 1)) & jnp.uint32(0xFFFF0000)
                acc0 = acc0 + plsc.bitcast(u0, jnp.float32) * wv[k]
                acc1 = acc1 + plsc.bitcast(u1, jnp.float32) * wv[k + 1]
            return jnp.sum(acc0 + acc1)

        for g in range(groups):
            def ebody(el, vout, g=g):
                e = g * 16 + 2 * el
                t0 = edge_dot(e)
                t1 = edge_dot(e + 1)
                vout = jnp.where(lane == 2 * el, t0, vout)
                return jnp.where(lane == 2 * el + 1, t1, vout)

            out_v[pl.ds(g * 16, 16)] = lax.fori_loop(0, 8, ebody, zero)

    pltpu.async_copy(a_hbm.at[src_v.at[0]], abuf0, sa0)
    pltpu.async_copy(b_hbm.at[dst_v.at[0]], bbuf0, sb0)

    def body(jj, carry):
        j = 2 * jj
        c1 = pltpu.async_copy(a_hbm.at[src_v.at[j + 1]], abuf1, sa1)
        c2 = pltpu.async_copy(b_hbm.at[dst_v.at[j + 1]], bbuf1, sb1)
        pltpu.make_async_copy(a_hbm.at[src_v.at[j]], abuf0, sa0).wait()
        pltpu.make_async_copy(b_hbm.at[dst_v.at[j]], bbuf0, sb0).wait()
        compute(abuf0, bbuf0)
        pltpu.sync_copy(out_v, out.at[base + j])

        @pl.when(jj + 1 < chunks // 2)
        def _():
            pltpu.async_copy(a_hbm.at[src_v.at[j + 2]], abuf0, sa0)
            pltpu.async_copy(b_hbm.at[dst_v.at[j + 2]], bbuf0, sb0)

        c1.wait()
        c2.wait()
        compute(abuf1, bbuf1)
        pltpu.sync_copy(out_v, out.at[base + j + 1])
        return carry

    lax.fori_loop(0, chunks // 2, body, 0)


def _sc_policy(a_nodes, b_nodes, srcp, dstp, w2):
    rows, ch = srcp.shape
    chunks = rows // 32
    f = pl.kernel(
        functools.partial(_sc_policy_body, chunks=chunks, ch=ch),
        out_type=jax.ShapeDtypeStruct((rows, ch), jnp.float32),
        mesh=plsc.VectorSubcoreMesh(core_axis_name="c", subcore_axis_name="s"),
        compiler_params=pltpu.CompilerParams(needs_layout_passes=False),
        scratch_types=[
            pltpu.VMEM((chunks, ch), jnp.int32),
            pltpu.VMEM((chunks, ch), jnp.int32),
            pltpu.VMEM((ch, 256), jnp.float32),
            pltpu.VMEM((ch, 256), jnp.float32),
            pltpu.VMEM((ch, 256), jnp.float32),
            pltpu.VMEM((ch, 256), jnp.float32),
            pltpu.VMEM((ch,), jnp.float32),
            pltpu.VMEM((256,), jnp.float32),
            pltpu.SemaphoreType.DMA,
            pltpu.SemaphoreType.DMA,
            pltpu.SemaphoreType.DMA,
            pltpu.SemaphoreType.DMA,
        ],
    )
    return f(a_nodes, b_nodes, srcp, dstp, w2)


def _gin_dense_body(s3, w, b, g, beta, out3, *, relu):
    s = jnp.concatenate([s3[0], s3[1]], axis=-1)
    h = lax.dot_general(s, w[...], (((1,), (1,)), ((), ())),
                        preferred_element_type=jnp.float32) + b[...]
    mu = jnp.mean(h, axis=-1, keepdims=True)
    var = jnp.mean((h - mu) ** 2, axis=-1, keepdims=True)
    h = (h - mu) * lax.rsqrt(var + _LN_EPS) * g[...] + beta[...]
    if relu:
        h = jnp.maximum(h, 0.0)
    out3[0] = h[:, :128]
    out3[1] = h[:, 128:]


def _gin_dense(s3, w, b, g, beta, *, nreal, relu, bn=1000):
    n = s3.shape[1]
    d = 2 * s3.shape[2]
    nb = nreal // bn
    return pl.pallas_call(
        functools.partial(_gin_dense_body, relu=relu),
        grid=(nb,),
        in_specs=[
            pl.BlockSpec((2, bn, d // 2), lambda i: (0, i, 0)),
            pl.BlockSpec((d, d), lambda i: (0, 0)),
            pl.BlockSpec((1, d), lambda i: (0, 0)),
            pl.BlockSpec((1, d), lambda i: (0, 0)),
            pl.BlockSpec((1, d), lambda i: (0, 0)),
        ],
        out_specs=pl.BlockSpec((2, bn, d // 2), lambda i: (0, i, 0)),
        out_shape=jax.ShapeDtypeStruct((2, n, d // 2), jnp.float32),
        compiler_params=pltpu.CompilerParams(
            dimension_semantics=("arbitrary",)),
    )(s3, w, b.reshape(1, d), g.reshape(1, d), beta.reshape(1, d))


def _finale_body(s3, xres3, w, b, g, beta, p1a, p1b, p1bias, batch_r,
                 a_out, b_out, pool_out, cnt_out, *, nseg):
    i = pl.program_id(0)
    s = jnp.concatenate([s3[0], s3[1]], axis=-1)
    res = jnp.concatenate([xres3[0], xres3[1]], axis=-1)
    h = lax.dot_general(s, w[...], (((1,), (1,)), ((), ())),
                        preferred_element_type=jnp.float32) + b[...]
    mu = jnp.mean(h, axis=-1, keepdims=True)
    var = jnp.mean((h - mu) ** 2, axis=-1, keepdims=True)
    h = (h - mu) * lax.rsqrt(var + _LN_EPS) * g[...] + beta[...]
    h = jnp.maximum(h + res, 0.0)
    a_out[...] = lax.dot_general(
        h, p1a[...], (((1,), (1,)), ((), ())),
        preferred_element_type=jnp.float32) + p1bias[...]
    b_out[...] = lax.dot_general(
        h, p1b[...], (((1,), (1,)), ((), ())),
        preferred_element_type=jnp.float32)
    bid = batch_r[0, 0, :]
    oh = (bid[:, None] == lax.broadcasted_iota(jnp.int32, (bid.shape[0], nseg), 1)
          ).astype(jnp.float32)
    poolc = lax.dot_general(oh, h, (((0,), (0,)), ((), ())),
                            preferred_element_type=jnp.float32,
                            precision=lax.Precision.HIGHEST)
    cntc = jnp.broadcast_to(jnp.sum(oh, axis=0)[:, None], poolc.shape)

    @pl.when(i == 0)
    def _init():
        pool_out[...] = jnp.zeros_like(pool_out)
        cnt_out[...] = jnp.zeros_like(cnt_out)

    pool_out[...] += poolc
    cnt_out[...] += cntc


def _finale(s3, xres3, w, b, g, beta, p1a, p1b, p1bias, batch,
            *, nseg, nreal, bn=1000):
    d = 2 * s3.shape[2]
    nb = nreal // bn
    batch_r = batch.reshape(nb, 1, bn)
    return pl.pallas_call(
        functools.partial(_finale_body, nseg=nseg),
        grid=(nb,),
        in_specs=[
            pl.BlockSpec((2, bn, d // 2), lambda i: (0, i, 0)),
            pl.BlockSpec((2, bn, d // 2), lambda i: (0, i, 0)),
            pl.BlockSpec((d, d), lambda i: (0, 0)),
            pl.BlockSpec((1, d), lambda i: (0, 0)),
            pl.BlockSpec((1, d), lambda i: (0, 0)),
            pl.BlockSpec((1, d), lambda i: (0, 0)),
            pl.BlockSpec((d, d), lambda i: (0, 0)),
            pl.BlockSpec((d, d), lambda i: (0, 0)),
            pl.BlockSpec((1, d), lambda i: (0, 0)),
            pl.BlockSpec((1, 1, bn), lambda i: (i, 0, 0)),
        ],
        out_specs=[
            pl.BlockSpec((bn, d), lambda i: (i, 0)),
            pl.BlockSpec((bn, d), lambda i: (i, 0)),
            pl.BlockSpec((nseg, d), lambda i: (0, 0)),
            pl.BlockSpec((nseg, d), lambda i: (0, 0)),
        ],
        out_shape=[
            jax.ShapeDtypeStruct((nreal, d), jnp.float32),
            jax.ShapeDtypeStruct((nreal, d), jnp.float32),
            jax.ShapeDtypeStruct((nseg, d), jnp.float32),
            jax.ShapeDtypeStruct((nseg, d), jnp.float32),
        ],
        compiler_params=pltpu.CompilerParams(
            dimension_semantics=("arbitrary",)),
    )(s3, xres3, w, b.reshape(1, d), g.reshape(1, d), beta.reshape(1, d),
      p1a, p1b, p1bias.reshape(1, d), batch_r)


def _value_body(pool, cnt, w1, b1, g1, be1, w2, b2, g2, be2, vw, vb, out):
    ge = pool[...] / jnp.maximum(cnt[...], 1.0)

    def fc_bn_relu(v, w, b, g, be):
        v = lax.dot_general(v, w[...], (((1,), (1,)), ((), ())),
                            preferred_element_type=jnp.float32) + b[...]
        mu = jnp.mean(v, axis=0, keepdims=True)
        var = jnp.mean((v - mu) ** 2, axis=0, keepdims=True)
        v = (v - mu) * lax.rsqrt(var + _BN_EPS) * g[...] + be[...]
        return jnp.maximum(v, 0.0)

    v = fc_bn_relu(ge, w1, b1, g1, be1)
    v = fc_bn_relu(v, w2, b2, g2, be2)
    vbf = v.astype(jnp.bfloat16).astype(jnp.float32)
    wbf = vw[...].astype(jnp.bfloat16).astype(jnp.float32)
    val = jnp.tanh(jnp.sum(vbf * wbf, axis=-1, keepdims=True) + vb[0, 0])
    out[...] = jnp.broadcast_to(val, out.shape)


def _value_head(pool, cnt, w1, b1, g1, be1, w2, b2, g2, be2, vw, vb):
    nseg, d = pool.shape
    full = lambda s: pl.BlockSpec(s, lambda: tuple(0 for _ in s))
    out = pl.pallas_call(
        _value_body,
        in_specs=[
            full((nseg, d)), full((nseg, d)),
            full((d, d)), full((1, d)), full((1, d)), full((1, d)),
            full((d, d)), full((1, d)), full((1, d)), full((1, d)),
            full((1, d)), full((1, 1)),
        ],
        out_specs=full((nseg, 128)),
        out_shape=jax.ShapeDtypeStruct((nseg, 128), jnp.float32),
    )(pool, cnt, w1, b1.reshape(1, d), g1.reshape(1, d), be1.reshape(1, d),
      w2, b2.reshape(1, d), g2.reshape(1, d), be2.reshape(1, d),
      vw, vb.reshape(1, 1))
    return out[:, :1]


def _to_split(h):
    n, d = h.shape
    return h.reshape(n, 2, d // 2).transpose(1, 0, 2)


def kernel(x, gin1_W, gin1_b, ln1_g, ln1_b, gin2_W, gin2_b, ln2_g, ln2_b,
           gin3_W, gin3_b, ln3_g, ln3_b, fc1_W, fc1_b, bn1_g, bn1_b,
           fc2_W, fc2_b, bn2_g, bn2_b, pe1_W, pe1_b, pe2_W, pe2_b,
           vh_W, vh_b, edge_index, batch):
    n, d = x.shape
    e = edge_index.shape[1]
    nseg = 64
    src = edge_index[0]
    dst = edge_index[1]

    rows = e // _CH
    srcg = jnp.stack([src, src + _NPAD]).reshape(2, rows, _CH)
    dstl = dst.reshape(rows, _CH)

    def agg(h3):
        s2f = _sc_agg(h3.reshape(2 * _NPAD, d // 2), srcg, dstl, n=_NPAD)
        return s2f.reshape(2, _NPAD, d // 2)

    x3 = jnp.zeros((2, _NPAD, d // 2), jnp.float32).at[:, :n].set(_to_split(x))
    h3 = _gin_dense(agg(x3), gin1_W, gin1_b, ln1_g, ln1_b, nreal=n, relu=True)
    h3 = _gin_dense(agg(h3), gin2_W, gin2_b, ln2_g, ln2_b, nreal=n, relu=True)
    p1a = pe1_W[:, :d]
    p1b = pe1_W[:, d:]
    a_nodes, b_nodes, pool, cnt = _finale(
        agg(h3), x3, gin3_W, gin3_b, ln3_g, ln3_b,
        p1a, p1b, pe1_b, batch, nseg=nseg, nreal=n)

    value = _value_head(pool, cnt, fc1_W, fc1_b, bn1_g, bn1_b,
                        fc2_W, fc2_b, bn2_g, bn2_b, vh_W, vh_b)

    ch2 = 64
    epad = 32 * 80 * ch2
    srcp = jnp.zeros((epad,), jnp.int32).at[:e].set(src).reshape(epad // ch2, ch2)
    dstp = jnp.zeros((epad,), jnp.int32).at[:e].set(dst).reshape(epad // ch2, ch2)
    logits_pad = _sc_policy(a_nodes, b_nodes, srcp, dstp, pe2_W[0])
    policy_logits = logits_pad.reshape(-1)[:e] + pe2_b[0]
    return (policy_logits, value)
